# baseline (speedup 1.0000x reference)
import functools

import jax
import jax.numpy as jnp
from jax import lax
from jax.experimental import pallas as pl
from jax.experimental.pallas import tpu as pltpu
from jax.experimental.pallas import tpu_sc as plsc

_L = 16


@functools.lru_cache(maxsize=None)
def _build(B, V, NN, D, n_cores, n_subcores):
    NW = n_cores * n_subcores
    G = 128
    per_w = B // NW
    NB = per_w // G
    assert per_w % G == 0 and D % _L == 0

    mesh = plsc.VectorSubcoreMesh(core_axis_name="c", subcore_axis_name="s")

    @functools.partial(
        pl.kernel,
        out_type=jax.ShapeDtypeStruct((B, D), jnp.float32),
        mesh=mesh,
        compiler_params=pltpu.CompilerParams(use_tc_tiling_on_sc=False),
        scratch_types=[
            pltpu.VMEM((G,), jnp.int32),
            pltpu.VMEM((G,), jnp.int32),
            pltpu.VMEM((G,), jnp.int32),
            pltpu.VMEM((G,), jnp.float32),
            pltpu.VMEM((G, D), jnp.float32),
            pltpu.VMEM((G, D), jnp.float32),
            pltpu.SemaphoreType.DMA,
            pltpu.SemaphoreType.DMA,
        ],
    )
    def k(ids_hbm, emb_hbm, comp_hbm, out_hbm,
          ids_v, idxm_v, idxc_v, mask_v, rows_a, rows_b, sem_a, sem_b):
        wid = lax.axis_index("s") * n_cores + lax.axis_index("c")
        base = wid * per_w

        def block(j, carry):
            row0 = base + j * G
            pltpu.sync_copy(ids_hbm.at[pl.ds(row0, G)], ids_v)
            for kk in range(G // _L):
                sl = pl.ds(kk * _L, _L)
                v = ids_v[sl]
                d = v - V
                keep = lax.shift_right_arithmetic(d, 31)
                idxm_v[sl] = jnp.bitwise_and(v, keep)
                idxc_v[sl] = jnp.bitwise_and(d, jnp.bitwise_not(keep))
                mask_v[sl] = (keep + 1).astype(jnp.float32)
            cp_a = pltpu.async_copy(emb_hbm.at[idxm_v], rows_a, sem_a)
            cp_b = pltpu.async_copy(comp_hbm.at[idxc_v], rows_b, sem_b)
            cp_a.wait()
            cp_b.wait()

            def blend_grp(t, carry2):
                rsl = pl.ds(t * _L, _L)
                m16 = mask_v[rsl]
                va = rows_a.at[rsl]
                vb = rows_b.at[rsl]
                for lane in range(_L):
                    lvec = jnp.full((_L,), lane, jnp.int32)
                    m = m16.at[lvec].get(mode="promise_in_bounds")
                    for c in range(D // _L):
                        sl = pl.ds(c * _L, _L)
                        a = va[lane, sl]
                        b = vb[lane, sl]
                        va[lane, sl] = a + m * (b - a)
                return carry2

            lax.fori_loop(0, G // _L, blend_grp, 0)

            pltpu.sync_copy(rows_a, out_hbm.at[pl.ds(row0, G)])
            return carry

        lax.fori_loop(0, NB, block, 0)

    return k


def kernel(input_ids, emb_weight, comp_weight):
    BATCH, HIST = input_ids.shape
    V, D = emb_weight.shape
    NN = comp_weight.shape[0]
    info = plsc.get_sparse_core_info()
    ids_flat = input_ids.reshape(-1).astype(jnp.int32)
    k = _build(BATCH * HIST, V, NN, D, info.num_cores, info.num_subcores)
    out = k(ids_flat, emb_weight, comp_weight)
    return out.reshape(BATCH, HIST, D)
